# Initial kernel scaffold; baseline (speedup 1.0000x reference)
import jax
import jax.numpy as jnp
from jax.experimental import pallas as pl


def kernel(user_list, input_embedding, session_lengths, mem):
    raise NotImplementedError("write your pallas kernel here")


# speedup vs baseline: 2.7119x; 2.7119x over previous
import jax
import jax.numpy as jnp
from jax import lax
from jax.experimental import pallas as pl
from jax.experimental.pallas import tpu as pltpu
from jax.experimental.pallas import tpu_sc as plsc

B, L, H, U, S = 4096, 20, 100, 1000, 15
SH = S * H
SHP = 1536
HP = 128
NC, NS = 2, 16
NW = NC * NS
BPW = B // NW
CHUNK = 16
NCH = BPW // CHUNK
UPW = 32
UPAD = NW * UPW
U_LAST = U - (NW - 1) * UPW
BC = 512
NSL = SH // 16
NFULL = (SH - H) // 16
NMEAN = H // 16


def _mean_body(x_ref, sl_ref, o_ref):
    m = jnp.sum(x_ref[...], axis=1) / sl_ref[...]
    o_ref[...] = jnp.pad(m, ((0, 0), (0, HP - H)))


def _mean_tc(input_embedding, session_lengths):
    blk = 512
    return pl.pallas_call(
        _mean_body,
        grid=(B // blk,),
        in_specs=[
            pl.BlockSpec((blk, L, H), lambda i: (i, 0, 0)),
            pl.BlockSpec((blk, 1), lambda i: (i, 0)),
        ],
        out_specs=pl.BlockSpec((blk, HP), lambda i: (i, 0)),
        out_shape=jax.ShapeDtypeStruct((B, HP), jnp.float32),
    )(input_embedding, session_lengths)


def _win_body(ul_ref, w_ref):
    i = pl.program_id(0)
    ulb = ul_ref[0, 0, :]
    u_grid = (lax.broadcasted_iota(jnp.int32, (BC, 8, 128), 1) * 128
              + lax.broadcasted_iota(jnp.int32, (BC, 8, 128), 2))
    b_grid = lax.broadcasted_iota(jnp.int32, (BC, 8, 128), 0) + i * BC
    cand = jnp.where(ulb[:, None, None] == u_grid, b_grid, -1)
    cmax = jnp.max(cand, axis=0)

    @pl.when(i == 0)
    def _():
        w_ref[...] = cmax

    @pl.when(i != 0)
    def _():
        w_ref[...] = jnp.maximum(w_ref[...], cmax)


def _win_tc(ul32):
    w = pl.pallas_call(
        _win_body,
        grid=(B // BC,),
        in_specs=[pl.BlockSpec((1, 1, BC), lambda i: (i, 0, 0))],
        out_specs=pl.BlockSpec((8, 128), lambda i: (0, 0)),
        out_shape=jax.ShapeDtypeStruct((8, 128), jnp.int32),
    )(ul32.reshape(B // BC, 1, BC))
    return w.reshape(UPAD)


def _repack(rows, flat):
    def row_body(r, carry):
        rbase = r * SH
        for k in range(NSL):
            flat[pl.ds(rbase + k * 16, 16)] = rows[r, pl.ds(k * 16, 16)]
        flat[pl.ds(rbase + SH - 16, 16)] = rows[r, pl.ds(SH - 16, 16)]
        return carry

    lax.fori_loop(0, CHUNK, row_body, 0)


def _gath_body(ul, memp, gath, ulv, rows_a, rows_b, flat_a, flat_b,
               sem_a, sem_b, sem_fa, sem_fb):
    cid = lax.axis_index("c")
    sid = lax.axis_index("s")
    wid = sid * NC + cid
    base = wid * BPW
    pltpu.sync_copy(ul.at[pl.ds(base, BPW)], ulv)

    def gath_chunk(c, buf, sem):
        idxv = ulv[pl.ds(c * CHUNK, 16)]
        return pltpu.async_copy(memp.at[idxv], buf, sem)

    cps = [gath_chunk(0, rows_a, sem_a), gath_chunk(1, rows_b, sem_b)]
    fcp = [None, None]
    for c in range(NCH):
        p = c % 2
        rows, flat = (rows_a, flat_a) if p == 0 else (rows_b, flat_b)
        rsem, fsem = (sem_a, sem_fa) if p == 0 else (sem_b, sem_fb)
        cps[p].wait()
        if fcp[p] is not None:
            fcp[p].wait()
        _repack(rows, flat)
        if c + 2 < NCH:
            cps[p] = gath_chunk(c + 2, rows, rsem)
        fcp[p] = pltpu.async_copy(
            flat, gath.at[pl.ds((base + c * CHUNK) * SH, CHUNK * SH)], fsem)
    fcp[0].wait()
    fcp[1].wait()


def _sc_gather(ul32, memp):
    mesh = plsc.VectorSubcoreMesh(core_axis_name="c", subcore_axis_name="s")
    f = pl.kernel(
        _gath_body,
        out_type=jax.ShapeDtypeStruct((B * SH,), jnp.float32),
        mesh=mesh,
        compiler_params=pltpu.CompilerParams(needs_layout_passes=False),
        scratch_types=[
            pltpu.VMEM((BPW,), jnp.int32),
            pltpu.VMEM((CHUNK, SHP), jnp.float32),
            pltpu.VMEM((CHUNK, SHP), jnp.float32),
            pltpu.VMEM((CHUNK * SH,), jnp.float32),
            pltpu.VMEM((CHUNK * SH,), jnp.float32),
            pltpu.SemaphoreType.DMA,
            pltpu.SemaphoreType.DMA,
            pltpu.SemaphoreType.DMA,
            pltpu.SemaphoreType.DMA,
        ],
    )
    return f(ul32, memp)


def _users_pass(blk, meanbuf, winv, nusers):
    iota = lax.iota(jnp.int32, 16)

    def user_body(ui, carry):
        wsplat = plsc.load_gather(winv, [jnp.full((16,), ui, jnp.int32)])
        occ = wsplat >= 0
        base = ui * SH
        for k in range(NFULL):
            cur = blk[pl.ds(base + k * 16, 16)]
            src = blk[pl.ds(base + H + k * 16, 16)]
            blk[pl.ds(base + k * 16, 16)] = jnp.where(occ, src, cur)
        d = blk[pl.ds(base + NFULL * 16 - 8, 16)]
        s = blk[pl.ds(base + SH - 16, 16)]
        tail = jnp.where(iota < 8, d, s)
        blk[pl.ds(base + NFULL * 16 - 8, 16)] = jnp.where(occ, tail, d)
        for k in range(NMEAN):
            cur = blk[pl.ds(base + SH - H + k * 16, 16)]
            m = meanbuf[ui, pl.ds(k * 16, 16)]
            blk[pl.ds(base + SH - H + k * 16, 16)] = jnp.where(occ, m, cur)
        d2 = blk[pl.ds(base + SH - 16, 16)]
        m2 = meanbuf[ui, pl.ds(H - 16, 16)]
        v = jnp.where(iota < 12, d2, m2)
        blk[pl.ds(base + SH - 16, 16)] = jnp.where(occ, v, d2)
        return carry

    lax.fori_loop(0, nusers, user_body, 0)


def _upd_body(win, mem1, meanp, memu, winv, widx, mblk, mblk_s, meanbuf, sem):
    cid = lax.axis_index("c")
    sid = lax.axis_index("s")
    wid = sid * NC + cid
    lo = wid * UPW

    pltpu.sync_copy(win.at[pl.ds(lo, UPW)], winv)
    wv0 = winv[pl.ds(0, 16)]
    wv1 = winv[pl.ds(16, 16)]
    widx[pl.ds(0, 16)] = jnp.maximum(wv0, 0)
    widx[pl.ds(16, 16)] = jnp.maximum(wv1, 0)
    cm = pltpu.async_copy(meanp.at[widx], meanbuf, sem)

    @pl.when(wid != NW - 1)
    def _():
        pltpu.sync_copy(mem1.at[pl.ds(lo * SH, UPW * SH)], mblk)

    @pl.when(wid == NW - 1)
    def _():
        pltpu.sync_copy(mem1.at[pl.ds(lo * SH, U_LAST * SH)], mblk_s)

    cm.wait()

    @pl.when(wid != NW - 1)
    def _():
        _users_pass(mblk, meanbuf, winv, UPW)
        pltpu.sync_copy(mblk, memu.at[pl.ds(lo * SH, UPW * SH)])

    @pl.when(wid == NW - 1)
    def _():
        _users_pass(mblk_s, meanbuf, winv, U_LAST)
        pltpu.sync_copy(mblk_s, memu.at[pl.ds(lo * SH, U_LAST * SH)])


def _sc_update(winners, mem1, meanp):
    mesh = plsc.VectorSubcoreMesh(core_axis_name="c", subcore_axis_name="s")
    f = pl.kernel(
        _upd_body,
        out_type=jax.ShapeDtypeStruct((U * SH,), jnp.float32),
        mesh=mesh,
        compiler_params=pltpu.CompilerParams(needs_layout_passes=False),
        scratch_types=[
            pltpu.VMEM((UPW,), jnp.int32),
            pltpu.VMEM((UPW,), jnp.int32),
            pltpu.VMEM((UPW * SH,), jnp.float32),
            pltpu.VMEM((U_LAST * SH,), jnp.float32),
            pltpu.VMEM((UPW, HP), jnp.float32),
            pltpu.SemaphoreType.DMA,
        ],
    )
    return f(winners, mem1, meanp)


def kernel(user_list, input_embedding, session_lengths, mem):
    ul32 = user_list.astype(jnp.int32)
    memp = jnp.pad(mem.reshape(U, SH), ((0, 0), (0, SHP - SH)))
    mean_p = _mean_tc(input_embedding, session_lengths)
    winners = _win_tc(ul32)
    gath = _sc_gather(ul32, memp)
    memu = _sc_update(winners, mem.reshape(U * SH), mean_p)
    return gath.reshape(B, S, H), mean_p[:, :H], memu.reshape(U, S, H)


# speedup vs baseline: 3.1703x; 1.1691x over previous
import jax
import jax.numpy as jnp
from jax import lax
from jax.experimental import pallas as pl
from jax.experimental.pallas import tpu as pltpu
from jax.experimental.pallas import tpu_sc as plsc

B, L, H, U, S = 4096, 20, 100, 1000, 15
HP = 128
SLAB = S * HP
NC, NS = 2, 16
NW = NC * NS
BPW = B // NW
CHUNK = 16
NCH = BPW // CHUNK
UPW = 32
UPAD = NW * UPW
U_LAST = U - (NW - 1) * UPW
BC = 512
NK = HP // 16

_SC_TILED = pltpu.CompilerParams(
    needs_layout_passes=False, use_tc_tiling_on_sc=True)
_SC_FLAT = pltpu.CompilerParams(needs_layout_passes=False)


def _pad_body(x_ref, o_ref):
    o_ref[...] = jnp.pad(x_ref[...], ((0, 0), (0, 0), (0, HP - H)))


def _pad_tc(mem):
    blk = 200
    return pl.pallas_call(
        _pad_body,
        grid=(U // blk,),
        in_specs=[pl.BlockSpec((blk, S, H), lambda i: (i, 0, 0))],
        out_specs=pl.BlockSpec((blk, S, HP), lambda i: (i, 0, 0)),
        out_shape=jax.ShapeDtypeStruct((U, S, HP), jnp.float32),
    )(mem)


def _mean_body(x_ref, sl_ref, o_ref):
    m = jnp.sum(x_ref[...], axis=1) / sl_ref[...]
    o_ref[...] = jnp.pad(m, ((0, 0), (0, HP - H)))


def _mean_tc(input_embedding, session_lengths):
    blk = 512
    return pl.pallas_call(
        _mean_body,
        grid=(B // blk,),
        in_specs=[
            pl.BlockSpec((blk, L, H), lambda i: (i, 0, 0)),
            pl.BlockSpec((blk, 1), lambda i: (i, 0)),
        ],
        out_specs=pl.BlockSpec((blk, HP), lambda i: (i, 0)),
        out_shape=jax.ShapeDtypeStruct((B, HP), jnp.float32),
    )(input_embedding, session_lengths)


def _win_body(ul_ref, w_ref):
    i = pl.program_id(0)
    ulb = ul_ref[0, 0, :]
    u_grid = (lax.broadcasted_iota(jnp.int32, (BC, 8, 128), 1) * 128
              + lax.broadcasted_iota(jnp.int32, (BC, 8, 128), 2))
    b_grid = lax.broadcasted_iota(jnp.int32, (BC, 8, 128), 0) + i * BC
    cand = jnp.where(ulb[:, None, None] == u_grid, b_grid, -1)
    cmax = jnp.max(cand, axis=0)

    @pl.when(i == 0)
    def _():
        w_ref[...] = cmax

    @pl.when(i != 0)
    def _():
        w_ref[...] = jnp.maximum(w_ref[...], cmax)


def _win_tc(ul32):
    w = pl.pallas_call(
        _win_body,
        grid=(B // BC,),
        in_specs=[pl.BlockSpec((1, 1, BC), lambda i: (i, 0, 0))],
        out_specs=pl.BlockSpec((8, 128), lambda i: (0, 0)),
        out_shape=jax.ShapeDtypeStruct((8, 128), jnp.int32),
    )(ul32.reshape(B // BC, 1, BC))
    return w.reshape(UPAD)


def _gath_body(ul, memp, gout, ulv, rows_a, rows_b, sem_a, sem_b,
               sem_fa, sem_fb):
    cid = lax.axis_index("c")
    sid = lax.axis_index("s")
    wid = sid * NC + cid
    base = wid * BPW
    pltpu.sync_copy(ul.at[pl.ds(base, BPW)], ulv)

    def gath_chunk(c, buf, sem):
        idxv = ulv[pl.ds(c * CHUNK, 16)]
        return pltpu.async_copy(memp.at[idxv], buf, sem)

    cps = [gath_chunk(0, rows_a, sem_a), gath_chunk(1, rows_b, sem_b)]
    for c in range(NCH):
        p = c % 2
        rows = rows_a if p == 0 else rows_b
        rsem, fsem = (sem_a, sem_fa) if p == 0 else (sem_b, sem_fb)
        cps[p].wait()
        fcp = pltpu.async_copy(
            rows, gout.at[pl.ds(base + c * CHUNK, CHUNK)], fsem)
        fcp.wait()
        if c + 2 < NCH:
            cps[p] = gath_chunk(c + 2, rows, rsem)


def _sc_gather(ul32, memp):
    mesh = plsc.VectorSubcoreMesh(core_axis_name="c", subcore_axis_name="s")
    f = pl.kernel(
        _gath_body,
        out_type=jax.ShapeDtypeStruct((B, S, HP), jnp.float32),
        mesh=mesh,
        compiler_params=_SC_TILED,
        scratch_types=[
            pltpu.VMEM((BPW,), jnp.int32),
            pltpu.VMEM((CHUNK, S, HP), jnp.float32),
            pltpu.VMEM((CHUNK, S, HP), jnp.float32),
            pltpu.SemaphoreType.DMA,
            pltpu.SemaphoreType.DMA,
            pltpu.SemaphoreType.DMA,
            pltpu.SemaphoreType.DMA,
        ],
    )
    return f(ul32, memp)


def _users_pass(blk, meanbuf, winv, nusers):
    def user_body(ui, carry):
        wsplat = plsc.load_gather(winv, [jnp.full((16,), ui, jnp.int32)])
        occ = wsplat >= 0
        base = ui * SLAB
        for s in range(S - 1):
            for k in range(NK):
                d = base + s * HP + k * 16
                cur = blk[pl.ds(d, 16)]
                src = blk[pl.ds(d + HP, 16)]
                blk[pl.ds(d, 16)] = jnp.where(occ, src, cur)
        for k in range(NK):
            d = base + (S - 1) * HP + k * 16
            cur = blk[pl.ds(d, 16)]
            m = meanbuf[ui, pl.ds(k * 16, 16)]
            blk[pl.ds(d, 16)] = jnp.where(occ, m, cur)
        return carry

    lax.fori_loop(0, nusers, user_body, 0)


def _upd_body(win, mem1, meanp, memu, winv, widx, mblk, mblk_s, meanbuf, sem):
    cid = lax.axis_index("c")
    sid = lax.axis_index("s")
    wid = sid * NC + cid
    lo = wid * UPW

    pltpu.sync_copy(win.at[pl.ds(lo, UPW)], winv)
    wv0 = winv[pl.ds(0, 16)]
    wv1 = winv[pl.ds(16, 16)]
    widx[pl.ds(0, 16)] = jnp.maximum(wv0, 0)
    widx[pl.ds(16, 16)] = jnp.maximum(wv1, 0)
    cm = pltpu.async_copy(meanp.at[widx], meanbuf, sem)

    @pl.when(wid != NW - 1)
    def _():
        pltpu.sync_copy(mem1.at[pl.ds(lo * SLAB, UPW * SLAB)], mblk)

    @pl.when(wid == NW - 1)
    def _():
        pltpu.sync_copy(mem1.at[pl.ds(lo * SLAB, U_LAST * SLAB)], mblk_s)

    cm.wait()

    @pl.when(wid != NW - 1)
    def _():
        _users_pass(mblk, meanbuf, winv, UPW)
        pltpu.sync_copy(mblk, memu.at[pl.ds(lo * SLAB, UPW * SLAB)])

    @pl.when(wid == NW - 1)
    def _():
        _users_pass(mblk_s, meanbuf, winv, U_LAST)
        pltpu.sync_copy(mblk_s, memu.at[pl.ds(lo * SLAB, U_LAST * SLAB)])


def _sc_update(winners, mem1, meanp):
    mesh = plsc.VectorSubcoreMesh(core_axis_name="c", subcore_axis_name="s")
    f = pl.kernel(
        _upd_body,
        out_type=jax.ShapeDtypeStruct((U * SLAB,), jnp.float32),
        mesh=mesh,
        compiler_params=_SC_FLAT,
        scratch_types=[
            pltpu.VMEM((UPW,), jnp.int32),
            pltpu.VMEM((UPW,), jnp.int32),
            pltpu.VMEM((UPW * SLAB,), jnp.float32),
            pltpu.VMEM((U_LAST * SLAB,), jnp.float32),
            pltpu.VMEM((UPW, HP), jnp.float32),
            pltpu.SemaphoreType.DMA,
        ],
    )
    return f(winners, mem1, meanp)


def kernel(user_list, input_embedding, session_lengths, mem):
    ul32 = user_list.astype(jnp.int32)
    memp = _pad_tc(mem)
    mean_p = _mean_tc(input_embedding, session_lengths)
    winners = _win_tc(ul32)
    gout = _sc_gather(ul32, memp)
    memu = _sc_update(winners, memp.reshape(U * SLAB), mean_p)
    return (gout[:, :, :H], mean_p[:, :H],
            memu.reshape(U, S, HP)[:, :, :H])
